# Initial kernel scaffold; baseline (speedup 1.0000x reference)
import jax
import jax.numpy as jnp
from jax.experimental import pallas as pl


def kernel(x, V, U):
    raise NotImplementedError("write your pallas kernel here")


# speedup vs baseline: 1.0369x; 1.0369x over previous
import functools

import jax
import jax.numpy as jnp
from jax import lax
from jax.experimental import pallas as pl
from jax.experimental.pallas import tpu as pltpu
from jax.experimental.pallas import tpu_sc as plsc

_NC = 2
_NS = 16
_NW = _NC * _NS

_CHUNK = 1024


def _project_body(v_ref, u_ref, w_ref):
    w_ref[...] = jnp.dot(v_ref[...], u_ref[...],
                         preferred_element_type=jnp.float32)


def _project(V, U):
    vocab, c = V.shape
    emb = U.shape[1]
    blk = 8000
    assert vocab % blk == 0
    return pl.pallas_call(
        _project_body,
        grid=(vocab // blk,),
        in_specs=[
            pl.BlockSpec((blk, c), lambda i: (i, 0)),
            pl.BlockSpec((c, emb), lambda i: (0, 0)),
        ],
        out_specs=pl.BlockSpec((blk, emb), lambda i: (i, 0)),
        out_shape=jax.ShapeDtypeStruct((vocab, emb), jnp.float32),
    )(V, U)


def _gather(W, idx):
    n = idx.shape[0]
    emb = W.shape[1]
    assert n % (_NW * _CHUNK) == 0
    b_per_w = n // _NW
    n_chunks = b_per_w // _CHUNK
    mesh = plsc.VectorSubcoreMesh(core_axis_name="c", subcore_axis_name="s")

    @functools.partial(
        pl.kernel,
        mesh=mesh,
        out_type=jax.ShapeDtypeStruct((n, emb), jnp.float32),
        compiler_params=pltpu.CompilerParams(use_tc_tiling_on_sc=False),
        scratch_types=[
            pltpu.VMEM((_CHUNK,), jnp.int32),
            pltpu.VMEM((_CHUNK, emb), jnp.float32),
            pltpu.SemaphoreType.DMA,
        ],
    )
    def k(w_hbm, idx_hbm, out_hbm, idx_v, rows_v, sem):
        wid = lax.axis_index("s") * _NC + lax.axis_index("c")
        base = wid * b_per_w

        def body(g, carry):
            off = pl.multiple_of(base + g * _CHUNK, _CHUNK)
            pltpu.sync_copy(idx_hbm.at[pl.ds(off, _CHUNK)], idx_v)
            pltpu.async_copy(w_hbm.at[idx_v], rows_v, sem).wait()
            pltpu.sync_copy(rows_v, out_hbm.at[pl.ds(off, _CHUNK)])
            return carry

        lax.fori_loop(0, n_chunks, body, 0)

    return k(W, idx)


def kernel(x, V, U):
    batch, hist = x.shape
    emb = U.shape[1]
    W = _project(V, U)
    out = _gather(W, x.reshape(-1).astype(jnp.int32))
    return out.reshape(batch, hist, emb)


# speedup vs baseline: 1.7096x; 1.6488x over previous
import functools

import jax
import jax.numpy as jnp
from jax import lax
from jax.experimental import pallas as pl
from jax.experimental.pallas import tpu as pltpu
from jax.experimental.pallas import tpu_sc as plsc

_NC = 2
_NS = 16
_NW = _NC * _NS
_PACK = 4
_VBLK = 2000


def _project_body(v0, v1, v2, v3, u_ref, w_ref):
    u = u_ref[...]
    w_ref[...] = jnp.concatenate(
        [
            jnp.dot(v0[...], u, preferred_element_type=jnp.float32),
            jnp.dot(v1[...], u, preferred_element_type=jnp.float32),
            jnp.dot(v2[...], u, preferred_element_type=jnp.float32),
            jnp.dot(v3[...], u, preferred_element_type=jnp.float32),
        ],
        axis=1,
    )


def _project_packed(V, U):
    vocab, c = V.shape
    emb = U.shape[1]
    assert emb * _PACK == 128 and vocab % (_PACK * _VBLK) == 0
    group = vocab // _PACK
    nblk = group // _VBLK
    vspec = lambda a: pl.BlockSpec((_VBLK, c), lambda i, a=a: (i + nblk * a, 0))
    return pl.pallas_call(
        _project_body,
        grid=(nblk,),
        in_specs=[vspec(0), vspec(1), vspec(2), vspec(3),
                  pl.BlockSpec((c, emb), lambda i: (0, 0))],
        out_specs=pl.BlockSpec((_VBLK, _PACK * emb), lambda i: (i, 0)),
        out_shape=jax.ShapeDtypeStruct((group, _PACK * emb), jnp.float32),
    )(V, V, V, V, U)


def _gather_transpose(W, xmT, emb):
    hist, batch = xmT.shape
    n_rows = W.shape[0]
    bw = batch // _NW
    mesh = plsc.VectorSubcoreMesh(core_axis_name="c", subcore_axis_name="s")

    @functools.partial(
        pl.kernel,
        mesh=mesh,
        out_type=jax.ShapeDtypeStruct((hist, emb, batch), jnp.float32),
        compiler_params=pltpu.CompilerParams(
            use_tc_tiling_on_sc=False, needs_layout_passes=False),
        scratch_types=[
            pltpu.VMEM((bw,), jnp.int32),
            pltpu.VMEM((bw, emb), jnp.float32),
            pltpu.VMEM((emb, bw), jnp.float32),
            pltpu.SemaphoreType.DMA,
        ],
    )
    def k(w_hbm, idx_hbm, out_hbm, idx_v, rows_v, t_v, sem):
        wid = lax.axis_index("s") * _NC + lax.axis_index("c")
        b0 = wid * bw
        lane = lax.iota(jnp.int32, 16)

        def body(h, carry):
            pltpu.sync_copy(idx_hbm.at[h, pl.ds(b0, bw)], idx_v)
            pltpu.async_copy(w_hbm.at[idx_v], rows_v, sem).wait()
            for p in range(emb // 16):
                rows16 = lane + p * 16
                for j in range(bw):
                    val = rows_v[j, pl.ds(p * 16, 16)]
                    plsc.store_scatter(t_v, [rows16, jnp.full((16,), j, jnp.int32)], val)
            pltpu.sync_copy(t_v, out_hbm.at[h, :, pl.ds(b0, bw)])
            return carry

        lax.fori_loop(0, hist, body, 0)

    return k(W, xmT)


def kernel(x, V, U):
    batch, hist = x.shape
    vocab = V.shape[0]
    emb = U.shape[1]
    group = vocab // _PACK
    W2 = _project_packed(V, U)
    W = W2.reshape(vocab, emb)
    xi = x.astype(jnp.int32)
    a = xi // group
    xm = (xi - a * group) * _PACK + a
    xmT = xm.T
    out_p = _gather_transpose(W, xmT, emb)
    return out_p.transpose(2, 0, 1)


# speedup vs baseline: 2.4965x; 1.4603x over previous
import functools

import jax
import jax.numpy as jnp
from jax import lax
from jax.experimental import pallas as pl
from jax.experimental.pallas import tpu as pltpu
from jax.experimental.pallas import tpu_sc as plsc

_NC = 2
_NS = 16
_NW = _NC * _NS
_PACK = 4
_VBLK = 15872
_GRID = 63

_DN = (((0,), (0,)), ((), ()))


def _project_body(vt_ref, u_ref, w_ref):
    p = lax.dot_general(vt_ref[...], u_ref[...], _DN,
                        preferred_element_type=jnp.float32)
    q = _VBLK // _PACK
    w_ref[...] = jnp.concatenate([p[a * q:(a + 1) * q] for a in range(_PACK)],
                                 axis=1)


def _project_packed(Vt, U):
    c, vocab = Vt.shape
    emb = U.shape[1]
    assert emb * _PACK == 128
    q = _VBLK // _PACK
    return pl.pallas_call(
        _project_body,
        grid=(_GRID,),
        in_specs=[pl.BlockSpec((c, _VBLK), lambda i: (0, i)),
                  pl.BlockSpec((c, emb), lambda i: (0, 0))],
        out_specs=pl.BlockSpec((q, _PACK * emb), lambda i: (i, 0)),
        out_shape=jax.ShapeDtypeStruct((vocab // _PACK, _PACK * emb),
                                       jnp.float32),
    )(Vt, U)


def _gather_transpose(W, xmT, emb):
    hist, batch = xmT.shape
    bw = batch // _NW
    mesh = plsc.VectorSubcoreMesh(core_axis_name="c", subcore_axis_name="s")

    @functools.partial(
        pl.kernel,
        mesh=mesh,
        out_type=jax.ShapeDtypeStruct((hist, emb, batch), jnp.float32),
        compiler_params=pltpu.CompilerParams(
            use_tc_tiling_on_sc=False, needs_layout_passes=False),
        scratch_types=[
            pltpu.VMEM((2, bw), jnp.int32),
            pltpu.VMEM((2, bw, emb), jnp.float32),
            pltpu.VMEM((2, emb, bw), jnp.float32),
            pltpu.SemaphoreType.DMA,
            pltpu.SemaphoreType.DMA,
        ],
    )
    def k(w_hbm, idx_hbm, out_hbm, idx_v, rows_v, t_v, sem_g, sem_o):
        wid = lax.axis_index("s") * _NC + lax.axis_index("c")
        b0 = wid * bw
        lane = lax.iota(jnp.int32, 16)

        def fetch(h, par):
            pltpu.sync_copy(idx_hbm.at[h, pl.ds(b0, bw)], idx_v.at[par])
            pltpu.async_copy(w_hbm.at[idx_v.at[par]], rows_v.at[par], sem_g)

        fetch(0, 0)

        def body(h, carry):
            par = lax.rem(h, 2)

            @pl.when(h + 1 < hist)
            def _():
                fetch(h + 1, 1 - par)

            pltpu.make_async_copy(w_hbm.at[idx_v.at[0]], rows_v.at[0],
                                  sem_g).wait()

            @pl.when(h >= 2)
            def _():
                pltpu.make_async_copy(t_v.at[0],
                                      out_hbm.at[0, :, pl.ds(b0, bw)],
                                      sem_o).wait()

            for p in range(emb // 16):
                rows16 = lane + p * 16
                for j in range(bw):
                    val = rows_v[par, j, pl.ds(p * 16, 16)]
                    plsc.store_scatter(
                        t_v.at[par],
                        [rows16, jnp.full((16,), j, jnp.int32)], val)
            pltpu.async_copy(t_v.at[par], out_hbm.at[h, :, pl.ds(b0, bw)],
                             sem_o)
            return carry

        lax.fori_loop(0, hist, body, 0)
        pltpu.make_async_copy(t_v.at[0], out_hbm.at[0, :, pl.ds(b0, bw)],
                              sem_o).wait()
        pltpu.make_async_copy(t_v.at[0], out_hbm.at[0, :, pl.ds(b0, bw)],
                              sem_o).wait()

    return k(W, xmT)


def kernel(x, V, U):
    batch, hist = x.shape
    vocab = V.shape[0]
    emb = U.shape[1]
    bulk = _GRID * _VBLK
    q = _VBLK // _PACK

    W2 = _project_packed(V.T, U)
    tail = jnp.dot(V[bulk:], U, preferred_element_type=jnp.float32)
    W2 = lax.dynamic_update_slice(W2, tail.reshape(-1, _PACK * emb),
                                  (bulk // _PACK, 0))
    W = W2.reshape(vocab, emb)

    xi = x.astype(jnp.int32)
    j = xi // _VBLK
    w = xi - j * _VBLK
    a = w // q
    s = w - a * q
    xm = jnp.where(xi < bulk, (j * q + s) * _PACK + a, xi)
    xmT = xm.T

    out_p = _gather_transpose(W, xmT, emb)
    return out_p.transpose(2, 0, 1)


# speedup vs baseline: 2.5845x; 1.0352x over previous
import functools

import jax
import jax.numpy as jnp
from jax import lax
from jax.experimental import pallas as pl
from jax.experimental.pallas import tpu as pltpu
from jax.experimental.pallas import tpu_sc as plsc

_NC = 2
_NS = 16
_NW = _NC * _NS
_PACK = 4
_VBLK = 15872
_GRID = 63

_DN = (((0,), (0,)), ((), ()))


def _project_body(vt_ref, u_ref, w_ref):
    p = lax.dot_general(vt_ref[...], u_ref[...], _DN,
                        preferred_element_type=jnp.float32)
    q = _VBLK // _PACK
    w_ref[...] = jnp.concatenate([p[a * q:(a + 1) * q] for a in range(_PACK)],
                                 axis=1)


def _project_packed(Vt, U):
    c, vocab = Vt.shape
    emb = U.shape[1]
    assert emb * _PACK == 128
    q = _VBLK // _PACK
    return pl.pallas_call(
        _project_body,
        grid=(_GRID,),
        in_specs=[pl.BlockSpec((c, _VBLK), lambda i: (0, i)),
                  pl.BlockSpec((c, emb), lambda i: (0, 0))],
        out_specs=pl.BlockSpec((q, _PACK * emb), lambda i: (i, 0)),
        out_shape=jax.ShapeDtypeStruct((vocab // _PACK, _PACK * emb),
                                       jnp.float32),
    )(Vt, U)


def _gather_transpose(W, xmT, emb):
    hist, batch = xmT.shape
    bw = batch // _NW
    mesh = plsc.VectorSubcoreMesh(core_axis_name="c", subcore_axis_name="s")

    @functools.partial(
        pl.kernel,
        mesh=mesh,
        out_type=jax.ShapeDtypeStruct((hist, emb, batch), jnp.float32),
        compiler_params=pltpu.CompilerParams(
            use_tc_tiling_on_sc=False, needs_layout_passes=False),
        scratch_types=[
            pltpu.VMEM((hist, bw), jnp.int32),
            pltpu.VMEM((2, bw, emb), jnp.float32),
            pltpu.VMEM((2, emb, bw), jnp.float32),
            pltpu.SemaphoreType.DMA,
            pltpu.SemaphoreType.DMA,
        ],
    )
    def k(w_hbm, idx_hbm, out_hbm, idx_v, rows_v, t_v, sem_g, sem_o):
        wid = lax.axis_index("s") * _NC + lax.axis_index("c")
        b0 = wid * bw
        lane = lax.iota(jnp.int32, 16)

        pltpu.sync_copy(idx_hbm.at[:, pl.ds(b0, bw)], idx_v)

        def fetch(h, par):
            pltpu.async_copy(w_hbm.at[idx_v.at[h]], rows_v.at[par], sem_g)

        fetch(0, 0)

        def body(h, carry):
            par = lax.rem(h, 2)

            @pl.when(h + 1 < hist)
            def _():
                fetch(h + 1, 1 - par)

            pltpu.make_async_copy(w_hbm.at[idx_v.at[0]], rows_v.at[0],
                                  sem_g).wait()

            @pl.when(h >= 2)
            def _():
                pltpu.make_async_copy(t_v.at[0],
                                      out_hbm.at[0, :, pl.ds(b0, bw)],
                                      sem_o).wait()

            for p in range(emb // 16):
                rows16 = lane + p * 16
                for j in range(bw):
                    val = rows_v[par, j, pl.ds(p * 16, 16)]
                    plsc.store_scatter(
                        t_v.at[par],
                        [rows16, jnp.full((16,), j, jnp.int32)], val)
            pltpu.async_copy(t_v.at[par], out_hbm.at[h, :, pl.ds(b0, bw)],
                             sem_o)
            return carry

        lax.fori_loop(0, hist, body, 0)
        pltpu.make_async_copy(t_v.at[0], out_hbm.at[0, :, pl.ds(b0, bw)],
                              sem_o).wait()
        pltpu.make_async_copy(t_v.at[0], out_hbm.at[0, :, pl.ds(b0, bw)],
                              sem_o).wait()

    return k(W, xmT)


def kernel(x, V, U):
    batch, hist = x.shape
    vocab = V.shape[0]
    emb = U.shape[1]
    bulk = _GRID * _VBLK
    q = _VBLK // _PACK

    W2 = _project_packed(V.T, U)
    tail = jnp.dot(V[bulk:], U, preferred_element_type=jnp.float32)
    W2 = lax.dynamic_update_slice(W2, tail.reshape(-1, _PACK * emb),
                                  (bulk // _PACK, 0))
    W = W2.reshape(vocab, emb)

    xi = x.astype(jnp.int32)
    j = xi // _VBLK
    w = xi - j * _VBLK
    a = w // q
    s = w - a * q
    xm = jnp.where(xi < bulk, (j * q + s) * _PACK + a, xi)
    xmT = xm.T

    out_p = _gather_transpose(W, xmT, emb)
    return out_p.transpose(2, 0, 1)


# speedup vs baseline: 3.7869x; 1.4653x over previous
import functools

import jax
import jax.numpy as jnp
from jax import lax
from jax.experimental import pallas as pl
from jax.experimental.pallas import tpu as pltpu
from jax.experimental.pallas import tpu_sc as plsc

_NC = 2
_NS = 16
_NW = _NC * _NS
_PACK = 4
_VBLK = 15872
_GRID = 63

_DN = (((0,), (0,)), ((), ()))


def _project_body(vt_ref, u_ref, w_ref):
    p = lax.dot_general(vt_ref[...], u_ref[...], _DN,
                        preferred_element_type=jnp.float32)
    q = _VBLK // _PACK
    w_ref[...] = jnp.concatenate([p[a * q:(a + 1) * q] for a in range(_PACK)],
                                 axis=1)


def _project_packed(Vt, U):
    c, vocab = Vt.shape
    emb = U.shape[1]
    assert emb * _PACK == 128
    q = _VBLK // _PACK
    return pl.pallas_call(
        _project_body,
        grid=(_GRID,),
        in_specs=[pl.BlockSpec((c, _VBLK), lambda i: (0, i)),
                  pl.BlockSpec((c, emb), lambda i: (0, 0))],
        out_specs=pl.BlockSpec((q, _PACK * emb), lambda i: (i, 0)),
        out_shape=jax.ShapeDtypeStruct((vocab // _PACK, _PACK * emb),
                                       jnp.float32),
    )(Vt, U)


def _gather_transpose(W, xmT, emb):
    hist, batch = xmT.shape
    bw = batch // _NW
    mesh = plsc.VectorSubcoreMesh(core_axis_name="c", subcore_axis_name="s")

    @functools.partial(
        pl.kernel,
        mesh=mesh,
        out_type=jax.ShapeDtypeStruct((hist, emb, batch), jnp.float32),
        compiler_params=pltpu.CompilerParams(
            use_tc_tiling_on_sc=False, needs_layout_passes=False),
        scratch_types=[
            pltpu.VMEM((hist, bw), jnp.int32),
            pltpu.VMEM((2, bw, emb), jnp.float32),
            pltpu.VMEM((2, emb, bw + 1), jnp.float32),
            pltpu.SemaphoreType.DMA,
            pltpu.SemaphoreType.DMA,
        ],
    )
    def k(w_hbm, idx_hbm, out_hbm, idx_v, rows_v, t_v, sem_g, sem_o):
        wid = lax.axis_index("s") * _NC + lax.axis_index("c")
        b0 = wid * bw
        lane = lax.iota(jnp.int32, 16)

        pltpu.sync_copy(idx_hbm.at[:, pl.ds(b0, bw)], idx_v)

        def fetch(h, par):
            pltpu.async_copy(w_hbm.at[idx_v.at[h]], rows_v.at[par], sem_g)

        fetch(0, 0)

        def body(h, carry):
            par = lax.rem(h, 2)

            @pl.when(h + 1 < hist)
            def _():
                fetch(h + 1, 1 - par)

            pltpu.make_async_copy(w_hbm.at[idx_v.at[0]], rows_v.at[0],
                                  sem_g).wait()

            @pl.when(h >= 2)
            def _():
                pltpu.make_async_copy(t_v.at[0, :, pl.ds(0, bw)],
                                      out_hbm.at[0, :, pl.ds(b0, bw)],
                                      sem_o).wait()

            for p in range(emb // 16):
                rows16 = lane + p * 16
                for j in range(bw):
                    val = rows_v[par, j, pl.ds(p * 16, 16)]
                    plsc.store_scatter(
                        t_v.at[par],
                        [rows16, jnp.full((16,), j, jnp.int32)], val)
            pltpu.async_copy(t_v.at[par, :, pl.ds(0, bw)],
                             out_hbm.at[h, :, pl.ds(b0, bw)], sem_o)
            return carry

        lax.fori_loop(0, hist, body, 0)
        pltpu.make_async_copy(t_v.at[0, :, pl.ds(0, bw)],
                              out_hbm.at[0, :, pl.ds(b0, bw)], sem_o).wait()
        pltpu.make_async_copy(t_v.at[0, :, pl.ds(0, bw)],
                              out_hbm.at[0, :, pl.ds(b0, bw)], sem_o).wait()

    return k(W, xmT)


def kernel(x, V, U):
    batch, hist = x.shape
    vocab = V.shape[0]
    emb = U.shape[1]
    bulk = _GRID * _VBLK
    q = _VBLK // _PACK

    W2 = _project_packed(V.T, U)
    tail = jnp.dot(V[bulk:], U, preferred_element_type=jnp.float32)
    W2 = lax.dynamic_update_slice(W2, tail.reshape(-1, _PACK * emb),
                                  (bulk // _PACK, 0))
    W = W2.reshape(vocab, emb)

    xi = x.astype(jnp.int32)
    j = xi // _VBLK
    w = xi - j * _VBLK
    a = w // q
    s = w - a * q
    xm = jnp.where(xi < bulk, (j * q + s) * _PACK + a, xi)
    xmT = xm.T

    out_p = _gather_transpose(W, xmT, emb)
    return out_p.transpose(2, 0, 1)


# speedup vs baseline: 4.3225x; 1.1414x over previous
import functools

import jax
import jax.numpy as jnp
from jax import lax
from jax.experimental import pallas as pl
from jax.experimental.pallas import tpu as pltpu
from jax.experimental.pallas import tpu_sc as plsc

_NC = 2
_NS = 16
_NW = _NC * _NS
_PACK = 4
_VBLK = 15872
_GRID = 63

_DN = (((0,), (0,)), ((), ()))


def _project_body(v0, v1, v2, v3, u4_ref, w_ref):
    acc = lax.dot_general(v0[...], u4_ref[pl.ds(0, 64), :], _DN,
                          preferred_element_type=jnp.float32)
    for a, va in enumerate((v1, v2, v3), start=1):
        acc += lax.dot_general(va[...], u4_ref[pl.ds(64 * a, 64), :], _DN,
                               preferred_element_type=jnp.float32)
    w_ref[...] = acc


def _project_packed(Vt, U4):
    c, vocab = Vt.shape
    q = _VBLK // _PACK
    vspec = lambda a: pl.BlockSpec((c, q), lambda i, a=a: (0, _PACK * i + a))
    return pl.pallas_call(
        _project_body,
        grid=(_GRID,),
        in_specs=[vspec(0), vspec(1), vspec(2), vspec(3),
                  pl.BlockSpec((_PACK * c, 128), lambda i: (0, 0))],
        out_specs=pl.BlockSpec((q, 128), lambda i: (i, 0)),
        out_shape=jax.ShapeDtypeStruct((vocab // _PACK, 128), jnp.float32),
    )(Vt, Vt, Vt, Vt, U4)


def _gather_transpose(W, xmT, emb):
    hist, batch = xmT.shape
    bw = batch // _NW
    mesh = plsc.VectorSubcoreMesh(core_axis_name="c", subcore_axis_name="s")

    @functools.partial(
        pl.kernel,
        mesh=mesh,
        out_type=jax.ShapeDtypeStruct((hist, emb, batch), jnp.float32),
        compiler_params=pltpu.CompilerParams(
            use_tc_tiling_on_sc=False, needs_layout_passes=False),
        scratch_types=[
            pltpu.VMEM((hist, bw), jnp.int32),
            pltpu.VMEM((3, bw, emb), jnp.float32),
            pltpu.VMEM((2, emb, bw + 1), jnp.float32),
            pltpu.SemaphoreType.DMA,
            pltpu.SemaphoreType.DMA,
        ],
    )
    def k(w_hbm, idx_hbm, out_hbm, idx_v, rows_v, t_v, sem_g, sem_o):
        wid = lax.axis_index("s") * _NC + lax.axis_index("c")
        b0 = wid * bw
        lane = lax.iota(jnp.int32, 16)

        pltpu.sync_copy(idx_hbm.at[:, pl.ds(b0, bw)], idx_v)

        def fetch(h, par):
            pltpu.async_copy(w_hbm.at[idx_v.at[h]], rows_v.at[par], sem_g)

        fetch(0, 0)
        fetch(1, 1)

        def body(h, carry):
            par = lax.rem(h, 3)
            tpar = lax.rem(h, 2)

            @pl.when(h + 2 < hist)
            def _():
                fetch(h + 2, lax.rem(h + 2, 3))

            pltpu.make_async_copy(w_hbm.at[idx_v.at[0]], rows_v.at[0],
                                  sem_g).wait()

            @pl.when(h >= 2)
            def _():
                pltpu.make_async_copy(t_v.at[0, :, pl.ds(0, bw)],
                                      out_hbm.at[0, :, pl.ds(b0, bw)],
                                      sem_o).wait()

            for p in range(emb // 16):
                rows16 = lane + p * 16
                for j in range(bw):
                    val = rows_v[par, j, pl.ds(p * 16, 16)]
                    plsc.store_scatter(
                        t_v.at[tpar],
                        [rows16, jnp.full((16,), j, jnp.int32)], val)
            pltpu.async_copy(t_v.at[tpar, :, pl.ds(0, bw)],
                             out_hbm.at[h, :, pl.ds(b0, bw)], sem_o)
            return carry

        lax.fori_loop(0, hist, body, 0)
        pltpu.make_async_copy(t_v.at[0, :, pl.ds(0, bw)],
                              out_hbm.at[0, :, pl.ds(b0, bw)], sem_o).wait()
        pltpu.make_async_copy(t_v.at[0, :, pl.ds(0, bw)],
                              out_hbm.at[0, :, pl.ds(b0, bw)], sem_o).wait()

    return k(W, xmT)


def kernel(x, V, U):
    batch, hist = x.shape
    vocab = V.shape[0]
    emb = U.shape[1]
    bulk = _GRID * _VBLK
    q = _VBLK // _PACK

    U4 = jax.scipy.linalg.block_diag(U, U, U, U)
    W2 = _project_packed(V.T, U4)
    tail = jnp.dot(V[bulk:], U, preferred_element_type=jnp.float32)
    W2 = lax.dynamic_update_slice(W2, tail.reshape(-1, _PACK * emb),
                                  (bulk // _PACK, 0))
    W = W2.reshape(vocab, emb)

    xi = x.astype(jnp.int32)
    m = xi // q
    s = xi - m * q
    xm = jnp.where(xi < bulk, ((m >> 2) * q + s) * _PACK + (m & 3), xi)
    xmT = xm.T

    out_p = _gather_transpose(W, xmT, emb)
    return out_p.transpose(2, 0, 1)
